# baseline (speedup 1.0000x reference)
import jax
import jax.numpy as jnp
from jax import lax
from jax.experimental import pallas as pl
from jax.experimental.pallas import tpu as pltpu
from jax.experimental.pallas import tpu_sc as plsc

N_NODES = 10000
D = 128
NC, NS = 2, 16
NW = NC * NS
E = 320000
E_W = E // NW
DEG_CH = 128
E_W_PAD = 10240
N_ACC = 10240
ROWS_PER_TILE = N_ACC // NS

HALF_N = 5120
N_ACC2 = HALF_N
ROWS_PER_TILE2 = N_ACC2 // NS
N_G = N_NODES + 16
E_T = E // NS
CH = 128
NCHUNK = 160
E_T_PAD = NCHUNK * CH

_mesh = plsc.VectorSubcoreMesh(
    core_axis_name="c", subcore_axis_name="s", num_cores=NC, num_subcores=NS
)


DEG_LANES = 128
DEG_NCHUNK = E_W_PAD // DEG_CH


def _deg_body(dst_hbm, ones_hbm, zero1_hbm, out_hbm, idx_v, ones_v, acc_sh):
    c = lax.axis_index("c")
    s = lax.axis_index("s")
    wid = c * NS + s
    pltpu.sync_copy(zero1_hbm, acc_sh.at[pl.ds(s * ROWS_PER_TILE, ROWS_PER_TILE)])
    pltpu.sync_copy(ones_hbm, ones_v)
    pltpu.sync_copy(dst_hbm.at[wid], idx_v)
    plsc.subcore_barrier()

    def body(j, carry):
        pltpu.sync_copy(ones_v, acc_sh.at[idx_v.at[j]], add=True)
        return carry

    lax.fori_loop(0, DEG_NCHUNK, body, 0)
    plsc.subcore_barrier()
    pltpu.sync_copy(
        acc_sh.at[pl.ds(s * ROWS_PER_TILE, ROWS_PER_TILE)],
        out_hbm.at[c, pl.ds(s * ROWS_PER_TILE, ROWS_PER_TILE), :],
    )


_deg_kernel = pl.kernel(
    _deg_body,
    out_type=jax.ShapeDtypeStruct((NC, N_ACC, DEG_LANES), jnp.float32),
    mesh=_mesh,
    scratch_types=[
        pltpu.VMEM((DEG_NCHUNK, DEG_CH), jnp.int32),
        pltpu.VMEM((DEG_CH, DEG_LANES), jnp.float32),
        pltpu.VMEM_SHARED((N_ACC, DEG_LANES), jnp.float32),
    ],
)


HALF = NCHUNK // 2


NBUF = 2
NROUND = NCHUNK // NBUF


def _agg_body(
    g_hbm, src_hbm, dst_hbm, zero_hbm, out_hbm,
    idx_s_v, idx_d_v, r0, r1, acc_sh,
):
    rows = (r0, r1)
    c = lax.axis_index("c")
    s = lax.axis_index("s")
    pltpu.sync_copy(zero_hbm, acc_sh.at[pl.ds(s * ROWS_PER_TILE2, ROWS_PER_TILE2)])
    pltpu.sync_copy(src_hbm.at[c, s], idx_s_v)
    pltpu.sync_copy(dst_hbm.at[c, s], idx_d_v)
    plsc.subcore_barrier()

    def pipeline(gs0, gs1):
        gsem = (gs0, gs1)

        def gather(j, b):
            pltpu.async_copy(g_hbm.at[idx_s_v.at[j]], rows[b], gsem[b])

        def wait_gather(j, b):
            pltpu.make_async_copy(g_hbm.at[idx_s_v.at[j]], rows[b], gsem[b]).wait()

        def scatter(j, b):
            pltpu.sync_copy(rows[b], acc_sh.at[idx_d_v.at[j]], add=True)

        for b in range(NBUF):
            gather(b, b)

        def round_body(i, carry):
            j = NBUF * i
            for b in range(NBUF):
                wait_gather(j + b, b)
                scatter(j + b, b)
                gather(j + NBUF + b, b)
            return carry

        lax.fori_loop(0, NROUND - 1, round_body, 0)
        j = NCHUNK - NBUF
        for b in range(NBUF):
            wait_gather(j + b, b)
            scatter(j + b, b)

    pl.run_scoped(pipeline, pltpu.SemaphoreType.DMA, pltpu.SemaphoreType.DMA)
    plsc.subcore_barrier()
    pltpu.sync_copy(
        acc_sh.at[pl.ds(s * ROWS_PER_TILE2, ROWS_PER_TILE2)],
        out_hbm.at[c, pl.ds(s * ROWS_PER_TILE2, ROWS_PER_TILE2), :],
    )


_agg_kernel = pl.kernel(
    _agg_body,
    out_type=jax.ShapeDtypeStruct((NC, N_ACC2, D), jnp.float32),
    mesh=_mesh,
    scratch_types=[
        pltpu.VMEM((NCHUNK, CH), jnp.int32),
        pltpu.VMEM((NCHUNK, CH), jnp.int32),
        pltpu.VMEM((CH, D), jnp.float32),
        pltpu.VMEM((CH, D), jnp.float32),
        pltpu.VMEM_SHARED((N_ACC2, D), jnp.float32),
    ],
)


def _g1_body(x_ref, w_ref, degp_ref, g_ref, dinv_ref):
    deg = degp_ref[0, :N_NODES, 0:1] + degp_ref[1, :N_NODES, 0:1] + 1.0
    dinv = lax.rsqrt(deg)
    h = jnp.dot(x_ref[...], w_ref[...], preferred_element_type=jnp.float32)
    g_ref[:N_NODES, :] = h * dinv
    g_ref[N_NODES:, :] = jnp.zeros((N_G - N_NODES, D), jnp.float32)
    dinv_ref[...] = dinv


_g1_kernel = pl.pallas_call(
    _g1_body,
    out_shape=(
        jax.ShapeDtypeStruct((N_G, D), jnp.float32),
        jax.ShapeDtypeStruct((N_NODES, 1), jnp.float32),
    ),
)


def _layer_body(aggp_ref, g_ref, dinv_ref, b_ref, w_ref, gout_ref):
    agg = jnp.concatenate(
        [aggp_ref[0, :HALF_N, :], aggp_ref[1, : N_NODES - HALF_N, :]], axis=0
    )
    u = agg + g_ref[:N_NODES, :]
    z = jnp.maximum(u * dinv_ref[...] + b_ref[...], 0.0)
    h = jnp.dot(z, w_ref[...], preferred_element_type=jnp.float32)
    gout_ref[:N_NODES, :] = h * dinv_ref[...]
    gout_ref[N_NODES:, :] = jnp.zeros((N_G - N_NODES, D), jnp.float32)


_layer_kernel = pl.pallas_call(
    _layer_body,
    out_shape=jax.ShapeDtypeStruct((N_G, D), jnp.float32),
)


def _final_body(aggp_ref, g_ref, dinv_ref, b_ref, wfc_ref, bfc_ref, out_ref):
    agg = jnp.concatenate(
        [aggp_ref[0, :HALF_N, :], aggp_ref[1, : N_NODES - HALF_N, :]], axis=0
    )
    u = agg + g_ref[:N_NODES, :]
    z = jnp.maximum(u * dinv_ref[...] + b_ref[...], 0.0)
    out_ref[...] = jnp.dot(z, wfc_ref[...], preferred_element_type=jnp.float32) + bfc_ref[...]


_final_kernel = pl.pallas_call(
    _final_body,
    out_shape=jax.ShapeDtypeStruct((N_NODES, 1), jnp.float32),
)


def kernel(x, edge_index, W1, b1, W2, b2, Wfc, bfc):
    src = edge_index[0]
    dst = edge_index[1]
    dst_deg = jnp.pad(
        dst.reshape(NW, E_W), ((0, 0), (0, E_W_PAD - E_W)), constant_values=N_NODES
    ).reshape(NW, DEG_NCHUNK, DEG_CH)
    pad_t = E_T_PAD - E_T
    e_ids = jnp.arange(E, dtype=jnp.int32)
    zrow = N_NODES + (e_ids & 15)
    spread_dst = e_ids & 4095
    src_locals, dst_locals = [], []
    for core in range(NC):
        rel = dst - core * HALF_N
        ok = (rel >= 0) & (rel < HALF_N)
        src_locals.append(jnp.where(ok, src, zrow))
        dst_locals.append(jnp.where(ok, rel, spread_dst))
    src_p = jnp.pad(
        jnp.stack(src_locals).reshape(NC, NS, E_T),
        ((0, 0), (0, 0), (0, pad_t)),
        constant_values=N_NODES,
    ).reshape(NC, NS, NCHUNK, CH)
    dst_p = jnp.pad(
        jnp.stack(dst_locals).reshape(NC, NS, E_T),
        ((0, 0), (0, 0), (0, pad_t)),
    ).reshape(NC, NS, NCHUNK, CH)
    zeros2d = jnp.zeros((ROWS_PER_TILE2, D), jnp.float32)
    zeros_deg = jnp.zeros((ROWS_PER_TILE, DEG_LANES), jnp.float32)
    ones_deg = jnp.ones((DEG_CH, DEG_LANES), jnp.float32)

    degp = _deg_kernel(dst_deg, ones_deg, zeros_deg)
    g1, dinv = _g1_kernel(x, W1, degp)
    agg1 = _agg_kernel(g1, src_p, dst_p, zeros2d)
    g2 = _layer_kernel(agg1, g1, dinv, b1.reshape(1, D), W2)
    agg2 = _agg_kernel(g2, src_p, dst_p, zeros2d)
    out = _final_kernel(
        agg2, g2, dinv, b2.reshape(1, D), Wfc, bfc.reshape(1, 1)
    )
    return out.reshape(-1)


# speedup vs baseline: 1.6522x; 1.6522x over previous
import jax
import jax.numpy as jnp
from jax import lax
from jax.experimental import pallas as pl
from jax.experimental.pallas import tpu as pltpu
from jax.experimental.pallas import tpu_sc as plsc

N_NODES = 10000
D = 128
NC, NS = 2, 16
NW = NC * NS
E = 320000
E_W = E // NW
DEG_CH = 128
E_W_PAD = 10240
N_ACC = 10240
ROWS_PER_TILE = N_ACC // NS

HALF_N = 5120
N_ACC2 = HALF_N + 8 * NS
ROWS_PER_TILE2 = N_ACC2 // NS
N_G = N_NODES + 16
E_T = E // NS
CH = 128
NCHUNK = 160
E_T_PAD = NCHUNK * CH

_mesh = plsc.VectorSubcoreMesh(
    core_axis_name="c", subcore_axis_name="s", num_cores=NC, num_subcores=NS
)


DEG_LANES = 128
DEG_NCHUNK = E_W_PAD // DEG_CH


def _deg_body(dst_hbm, ones_hbm, zero1_hbm, out_hbm, idx_v, ones_v, acc_sh):
    c = lax.axis_index("c")
    s = lax.axis_index("s")
    wid = c * NS + s
    pltpu.sync_copy(zero1_hbm, acc_sh.at[pl.ds(s * ROWS_PER_TILE, ROWS_PER_TILE)])
    pltpu.sync_copy(ones_hbm, ones_v)
    pltpu.sync_copy(dst_hbm.at[wid], idx_v)
    plsc.subcore_barrier()

    def body(j, carry):
        pltpu.sync_copy(ones_v, acc_sh.at[idx_v.at[j]], add=True)
        return carry

    lax.fori_loop(0, DEG_NCHUNK, body, 0)
    plsc.subcore_barrier()
    pltpu.sync_copy(
        acc_sh.at[pl.ds(s * ROWS_PER_TILE, ROWS_PER_TILE)],
        out_hbm.at[c, pl.ds(s * ROWS_PER_TILE, ROWS_PER_TILE), :],
    )


_deg_kernel = pl.kernel(
    _deg_body,
    out_type=jax.ShapeDtypeStruct((NC, N_ACC, DEG_LANES), jnp.float32),
    mesh=_mesh,
    scratch_types=[
        pltpu.VMEM((DEG_NCHUNK, DEG_CH), jnp.int32),
        pltpu.VMEM((DEG_CH, DEG_LANES), jnp.float32),
        pltpu.VMEM_SHARED((N_ACC, DEG_LANES), jnp.float32),
    ],
)


HALF = NCHUNK // 2


NBUF = 2
NROUND = NCHUNK // NBUF


def _agg_body(
    g_hbm, src_hbm, dst_hbm, zero_hbm, out_hbm,
    idx_s_v, idx_d_v, r0, r1, acc_sh,
):
    rows = (r0, r1)
    c = lax.axis_index("c")
    s = lax.axis_index("s")
    pltpu.sync_copy(zero_hbm, acc_sh.at[pl.ds(s * ROWS_PER_TILE2, ROWS_PER_TILE2)])
    pltpu.sync_copy(src_hbm.at[s], idx_s_v)
    pltpu.sync_copy(dst_hbm.at[c, s], idx_d_v)
    plsc.subcore_barrier()

    def pipeline(gs0, gs1, ss0, ss1):
        gsem = (gs0, gs1)
        ssem = (ss0, ss1)

        def gather(j, b):
            pltpu.async_copy(g_hbm.at[idx_s_v.at[j]], rows[b], gsem[b])

        def wait_gather(j, b):
            pltpu.make_async_copy(g_hbm.at[idx_s_v.at[j]], rows[b], gsem[b]).wait()

        def scatter(j, b):
            pltpu.async_copy(rows[b], acc_sh.at[idx_d_v.at[j]], ssem[b], add=True)

        def wait_scatter(j, b):
            pltpu.make_async_copy(rows[b], acc_sh.at[idx_d_v.at[j]], ssem[b]).wait()

        for b in range(NBUF):
            gather(b, b)

        def round_body(i, carry):
            j = NBUF * i
            for b in range(NBUF):
                wait_gather(j + b, b)
                scatter(j + b, b)
            for b in range(NBUF):
                wait_scatter(j + b, b)
                gather(j + NBUF + b, b)
            return carry

        lax.fori_loop(0, NROUND - 1, round_body, 0)
        j = NCHUNK - NBUF
        for b in range(NBUF):
            wait_gather(j + b, b)
            scatter(j + b, b)
        for b in range(NBUF):
            wait_scatter(j + b, b)

    pl.run_scoped(
        pipeline,
        pltpu.SemaphoreType.DMA,
        pltpu.SemaphoreType.DMA,
        pltpu.SemaphoreType.DMA,
        pltpu.SemaphoreType.DMA,
    )
    plsc.subcore_barrier()
    pltpu.sync_copy(
        acc_sh.at[pl.ds(s * ROWS_PER_TILE2, ROWS_PER_TILE2)],
        out_hbm.at[c, pl.ds(s * ROWS_PER_TILE2, ROWS_PER_TILE2), :],
    )


_agg_kernel = pl.kernel(
    _agg_body,
    out_type=jax.ShapeDtypeStruct((NC, N_ACC2, D), jnp.float32),
    mesh=_mesh,
    scratch_types=[
        pltpu.VMEM((NCHUNK, CH), jnp.int32),
        pltpu.VMEM((NCHUNK, CH), jnp.int32),
        pltpu.VMEM((CH, D), jnp.float32),
        pltpu.VMEM((CH, D), jnp.float32),
        pltpu.VMEM_SHARED((N_ACC2, D), jnp.float32),
    ],
)


def _g1_body(x_ref, w_ref, degp_ref, g_ref, dinv_ref):
    deg = degp_ref[0, :N_NODES, 0:1] + degp_ref[1, :N_NODES, 0:1] + 1.0
    dinv = lax.rsqrt(deg)
    h = jnp.dot(x_ref[...], w_ref[...], preferred_element_type=jnp.float32)
    g_ref[:N_NODES, :] = h * dinv
    g_ref[N_NODES:, :] = jnp.zeros((N_G - N_NODES, D), jnp.float32)
    dinv_ref[...] = dinv


_g1_kernel = pl.pallas_call(
    _g1_body,
    out_shape=(
        jax.ShapeDtypeStruct((N_G, D), jnp.float32),
        jax.ShapeDtypeStruct((N_NODES, 1), jnp.float32),
    ),
)


def _layer_body(aggp_ref, g_ref, dinv_ref, b_ref, w_ref, gout_ref):
    agg = jnp.concatenate(
        [aggp_ref[0, :HALF_N, :], aggp_ref[1, : N_NODES - HALF_N, :]], axis=0
    )
    u = agg + g_ref[:N_NODES, :]
    z = jnp.maximum(u * dinv_ref[...] + b_ref[...], 0.0)
    h = jnp.dot(z, w_ref[...], preferred_element_type=jnp.float32)
    gout_ref[:N_NODES, :] = h * dinv_ref[...]
    gout_ref[N_NODES:, :] = jnp.zeros((N_G - N_NODES, D), jnp.float32)


_layer_kernel = pl.pallas_call(
    _layer_body,
    out_shape=jax.ShapeDtypeStruct((N_G, D), jnp.float32),
)


def _final_body(aggp_ref, g_ref, dinv_ref, b_ref, wfc_ref, bfc_ref, out_ref):
    agg = jnp.concatenate(
        [aggp_ref[0, :HALF_N, :], aggp_ref[1, : N_NODES - HALF_N, :]], axis=0
    )
    u = agg + g_ref[:N_NODES, :]
    z = jnp.maximum(u * dinv_ref[...] + b_ref[...], 0.0)
    out_ref[...] = jnp.dot(z, wfc_ref[...], preferred_element_type=jnp.float32) + bfc_ref[...]


_final_kernel = pl.pallas_call(
    _final_body,
    out_shape=jax.ShapeDtypeStruct((N_NODES, 1), jnp.float32),
)


def kernel(x, edge_index, W1, b1, W2, b2, Wfc, bfc):
    src = edge_index[0]
    dst = edge_index[1]
    dst_deg = jnp.pad(
        dst.reshape(NW, E_W), ((0, 0), (0, E_W_PAD - E_W)), constant_values=N_NODES
    ).reshape(NW, DEG_NCHUNK, DEG_CH)
    pad_t = E_T_PAD - E_T
    e_ids = jnp.arange(E, dtype=jnp.int32)
    tile_of_e = e_ids // E_T
    dummy = HALF_N + tile_of_e * 8 + (e_ids & 7)
    dst_locals = []
    for core in range(NC):
        rel = dst - core * HALF_N
        ok = (rel >= 0) & (rel < HALF_N)
        dst_locals.append(jnp.where(ok, rel, dummy))
    src_p = jnp.pad(
        src.reshape(NS, E_T), ((0, 0), (0, pad_t)), constant_values=N_NODES
    ).reshape(NS, NCHUNK, CH)
    dst_tail = (
        HALF_N
        + jnp.arange(NS, dtype=jnp.int32)[None, :, None] * 8
        + (jnp.arange(pad_t, dtype=jnp.int32)[None, None, :] & 7)
    )
    dst_p = jnp.concatenate(
        [
            jnp.stack(dst_locals).reshape(NC, NS, E_T),
            jnp.broadcast_to(dst_tail, (NC, NS, pad_t)),
        ],
        axis=2,
    ).reshape(NC, NS, NCHUNK, CH)
    zeros2d = jnp.zeros((ROWS_PER_TILE2, D), jnp.float32)
    zeros_deg = jnp.zeros((ROWS_PER_TILE, DEG_LANES), jnp.float32)
    ones_deg = jnp.ones((DEG_CH, DEG_LANES), jnp.float32)

    degp = _deg_kernel(dst_deg, ones_deg, zeros_deg)
    g1, dinv = _g1_kernel(x, W1, degp)
    agg1 = _agg_kernel(g1, src_p, dst_p, zeros2d)
    g2 = _layer_kernel(agg1, g1, dinv, b1.reshape(1, D), W2)
    agg2 = _agg_kernel(g2, src_p, dst_p, zeros2d)
    out = _final_kernel(
        agg2, g2, dinv, b2.reshape(1, D), Wfc, bfc.reshape(1, 1)
    )
    return out.reshape(-1)


# speedup vs baseline: 2.8519x; 1.7261x over previous
import jax
import jax.numpy as jnp
from jax import lax
from jax.experimental import pallas as pl
from jax.experimental.pallas import tpu as pltpu
from jax.experimental.pallas import tpu_sc as plsc

N_NODES = 10000
D = 128
NC, NS = 2, 16
NW = NC * NS
E = 320000
E_W = E // NW
CH = 128
NCHUNK = 80
E_W_PAD = NCHUNK * CH
N_ACC = 10240
ROWS_PER_TILE = N_ACC // NS
DEG_LANES = 128

_mesh = plsc.VectorSubcoreMesh(
    core_axis_name="c", subcore_axis_name="s", num_cores=NC, num_subcores=NS
)


def _deg_body(dst_hbm, ones_hbm, zero1_hbm, out_hbm, idx_v, ones_v, acc_sh):
    c = lax.axis_index("c")
    s = lax.axis_index("s")
    wid = c * NS + s
    pltpu.sync_copy(zero1_hbm, acc_sh.at[pl.ds(s * ROWS_PER_TILE, ROWS_PER_TILE)])
    pltpu.sync_copy(ones_hbm, ones_v)
    pltpu.sync_copy(dst_hbm.at[wid], idx_v)
    plsc.subcore_barrier()

    def body(j, carry):
        pltpu.sync_copy(ones_v, acc_sh.at[idx_v.at[j]], add=True)
        return carry

    lax.fori_loop(0, NCHUNK, body, 0)
    plsc.subcore_barrier()
    pltpu.sync_copy(
        acc_sh.at[pl.ds(s * ROWS_PER_TILE, ROWS_PER_TILE)],
        out_hbm.at[c, pl.ds(s * ROWS_PER_TILE, ROWS_PER_TILE), :],
    )


_deg_kernel = pl.kernel(
    _deg_body,
    out_type=jax.ShapeDtypeStruct((NC, N_ACC, DEG_LANES), jnp.float32),
    mesh=_mesh,
    scratch_types=[
        pltpu.VMEM((NCHUNK, CH), jnp.int32),
        pltpu.VMEM((CH, DEG_LANES), jnp.float32),
        pltpu.VMEM_SHARED((N_ACC, DEG_LANES), jnp.float32),
    ],
)


def _agg_body(g_hbm, src_hbm, dst_hbm, zero_hbm, out_hbm, idx_s_v, idx_d_v, rows_v, acc_sh):
    c = lax.axis_index("c")
    s = lax.axis_index("s")
    wid = c * NS + s
    pltpu.sync_copy(zero_hbm, acc_sh.at[pl.ds(s * ROWS_PER_TILE, ROWS_PER_TILE)])
    pltpu.sync_copy(src_hbm.at[wid], idx_s_v)
    pltpu.sync_copy(dst_hbm.at[wid], idx_d_v)
    plsc.subcore_barrier()

    def body(j, carry):
        pltpu.sync_copy(g_hbm.at[idx_s_v.at[j]], rows_v)
        pltpu.sync_copy(rows_v, acc_sh.at[idx_d_v.at[j]], add=True)
        return carry

    lax.fori_loop(0, NCHUNK, body, 0)
    plsc.subcore_barrier()
    pltpu.sync_copy(
        acc_sh.at[pl.ds(s * ROWS_PER_TILE, ROWS_PER_TILE)],
        out_hbm.at[c, pl.ds(s * ROWS_PER_TILE, ROWS_PER_TILE), :],
    )


_agg_kernel = pl.kernel(
    _agg_body,
    out_type=jax.ShapeDtypeStruct((NC, N_ACC, D), jnp.float32),
    mesh=_mesh,
    scratch_types=[
        pltpu.VMEM((NCHUNK, CH), jnp.int32),
        pltpu.VMEM((NCHUNK, CH), jnp.int32),
        pltpu.VMEM((CH, D), jnp.float32),
        pltpu.VMEM_SHARED((N_ACC, D), jnp.float32),
    ],
)


def _g1_body(x_ref, w_ref, degp_ref, g_ref, dinv_ref):
    deg = degp_ref[0, :N_NODES, 0:1] + degp_ref[1, :N_NODES, 0:1] + 1.0
    dinv = lax.rsqrt(deg)
    h = jnp.dot(x_ref[...], w_ref[...], preferred_element_type=jnp.float32)
    g_ref[...] = h * dinv
    dinv_ref[...] = dinv


_g1_kernel = pl.pallas_call(
    _g1_body,
    out_shape=(
        jax.ShapeDtypeStruct((N_NODES, D), jnp.float32),
        jax.ShapeDtypeStruct((N_NODES, 1), jnp.float32),
    ),
)


def _layer_body(aggp_ref, g_ref, dinv_ref, b_ref, w_ref, gout_ref):
    u = aggp_ref[0, :N_NODES, :] + aggp_ref[1, :N_NODES, :] + g_ref[...]
    z = jnp.maximum(u * dinv_ref[...] + b_ref[...], 0.0)
    h = jnp.dot(z, w_ref[...], preferred_element_type=jnp.float32)
    gout_ref[...] = h * dinv_ref[...]


_layer_kernel = pl.pallas_call(
    _layer_body,
    out_shape=jax.ShapeDtypeStruct((N_NODES, D), jnp.float32),
)


def _final_body(aggp_ref, g_ref, dinv_ref, b_ref, wfc_ref, bfc_ref, out_ref):
    u = aggp_ref[0, :N_NODES, :] + aggp_ref[1, :N_NODES, :] + g_ref[...]
    z = jnp.maximum(u * dinv_ref[...] + b_ref[...], 0.0)
    out_ref[...] = jnp.dot(z, wfc_ref[...], preferred_element_type=jnp.float32) + bfc_ref[...]


_final_kernel = pl.pallas_call(
    _final_body,
    out_shape=jax.ShapeDtypeStruct((N_NODES, 1), jnp.float32),
)


def kernel(x, edge_index, W1, b1, W2, b2, Wfc, bfc):
    src = edge_index[0].reshape(NW, E_W)
    dst = edge_index[1].reshape(NW, E_W)
    pad = E_W_PAD - E_W
    src_p = jnp.pad(src, ((0, 0), (0, pad))).reshape(NW, NCHUNK, CH)
    dst_p = jnp.pad(dst, ((0, 0), (0, pad)), constant_values=N_NODES).reshape(
        NW, NCHUNK, CH
    )
    zeros2d = jnp.zeros((ROWS_PER_TILE, D), jnp.float32)
    zeros_deg = jnp.zeros((ROWS_PER_TILE, DEG_LANES), jnp.float32)
    ones_deg = jnp.ones((CH, DEG_LANES), jnp.float32)

    degp = _deg_kernel(dst_p, ones_deg, zeros_deg)
    g1, dinv = _g1_kernel(x, W1, degp)
    agg1 = _agg_kernel(g1, src_p, dst_p, zeros2d)
    g2 = _layer_kernel(agg1, g1, dinv, b1.reshape(1, D), W2)
    agg2 = _agg_kernel(g2, src_p, dst_p, zeros2d)
    out = _final_kernel(
        agg2, g2, dinv, b2.reshape(1, D), Wfc, bfc.reshape(1, 1)
    )
    return out.reshape(-1)
